# Initial kernel scaffold; baseline (speedup 1.0000x reference)
import jax
import jax.numpy as jnp
from jax.experimental import pallas as pl


def kernel(x, edge_index, cand_idxs, W1, a_src1, a_dst1, W2, a_src2, a_dst2, w_score, b_score):
    raise NotImplementedError("write your pallas kernel here")


# speedup vs baseline: 18.6797x; 18.6797x over previous
import functools

import jax
import jax.numpy as jnp
from jax import lax
from jax.experimental import pallas as pl
from jax.experimental.pallas import tpu as pltpu
from jax.experimental.pallas import tpu_sc as plsc

N = 10000
E = 320000
D = 128
DH = D // 2
C = 32

L = 16
NC = 2
NS = 16
CHUNK = 80

EDGES_TILE = E // NS
ROWS = EDGES_TILE // CHUNK
QS = DH // L
OUT_ROWS_TILE = 624
OUT_ROWS_REM = N - NS * OUT_ROWS_TILE
OUT_REM_BASE = NS * OUT_ROWS_TILE

_NEG_SLOPE = 0.2


def _dense_common(x, W, a_s, a_d, hlo_ref, hhi_ref, als_ref, ald_ref):
    h = jnp.dot(x.astype(jnp.bfloat16), W.astype(jnp.bfloat16),
                preferred_element_type=jnp.float32)
    hlo_ref[...] = h[:, :DH]
    hhi_ref[...] = h[:, DH:]
    h16 = h.astype(jnp.bfloat16)
    als_ref[...] = jnp.dot(h16, a_s.astype(jnp.bfloat16),
                           preferred_element_type=jnp.float32)
    ald_ref[...] = jnp.dot(h16, a_d.astype(jnp.bfloat16),
                           preferred_element_type=jnp.float32)


def _dense1_body(x_ref, w_ref, as_ref, ad_ref, hlo_ref, hhi_ref, als_ref,
                 ald_ref):
    _dense_common(x_ref[...], w_ref[...], as_ref[...], ad_ref[...],
                  hlo_ref, hhi_ref, als_ref, ald_ref)


def _dense2_body(o_ref, w_ref, as_ref, ad_ref, hlo_ref, hhi_ref, als_ref,
                 ald_ref):
    t = jnp.maximum(jnp.concatenate([o_ref[0], o_ref[1]], axis=-1), 0.0)
    _dense_common(t, w_ref[...], as_ref[...], ad_ref[...],
                  hlo_ref, hhi_ref, als_ref, ald_ref)


_dense_out = [
    jax.ShapeDtypeStruct((N, DH), jnp.float32),
    jax.ShapeDtypeStruct((N, DH), jnp.float32),
    jax.ShapeDtypeStruct((N,), jnp.float32),
    jax.ShapeDtypeStruct((N,), jnp.float32),
]

_dense1 = pl.pallas_call(_dense1_body, out_shape=_dense_out)
_dense2 = pl.pallas_call(_dense2_body, out_shape=_dense_out)


_mesh = plsc.VectorSubcoreMesh(core_axis_name="c", subcore_axis_name="s")
_sc_params = pltpu.CompilerParams(needs_layout_passes=False,
                                  use_tc_tiling_on_sc=False)


@functools.partial(
    pl.kernel,
    out_type=jax.ShapeDtypeStruct((NC, N, DH), jnp.float32),
    mesh=_mesh,
    compiler_params=_sc_params,
    scratch_types=[
        pltpu.VMEM((N,), jnp.float32),
        pltpu.VMEM((N,), jnp.float32),
        pltpu.VMEM((N,), jnp.float32),
        pltpu.VMEM((EDGES_TILE,), jnp.int32),
        pltpu.VMEM((ROWS, CHUNK), jnp.int32),
        pltpu.VMEM((CHUNK, DH), jnp.float32),
        pltpu.VMEM((CHUNK,), jnp.float32),
        pltpu.VMEM_SHARED((N,), jnp.float32),
        pltpu.VMEM_SHARED((N, DH), jnp.float32),
    ],
)
def _edge_kernel(as_hbm, ad_hbm, hlo_hbm, hhi_hbm, src_hbm, dst_p1_hbm,
                 out_hbm, asv, adv, denv, srcv, dstv2, rowsv, pbuf,
                 den_sh, out_sh):
    cid = lax.axis_index("c")
    sid = lax.axis_index("s")
    zero16 = jnp.zeros((L,), jnp.float32)

    def _zrow(r, carry):
        for q in range(QS):
            rowsv[r, pl.ds(q * L, L)] = zero16
        return carry
    lax.fori_loop(0, CHUNK, _zrow, 0)

    def _zden(i, carry):
        denv[pl.ds(i * L, L)] = zero16
        return carry
    lax.fori_loop(0, N // L, _zden, 0)

    base = sid * OUT_ROWS_TILE
    nfull = OUT_ROWS_TILE // CHUNK
    rem = OUT_ROWS_TILE - nfull * CHUNK
    for k in range(nfull):
        pltpu.sync_copy(rowsv, out_sh.at[pl.ds(base + k * CHUNK, CHUNK)])
    pltpu.sync_copy(rowsv.at[pl.ds(0, rem)],
                    out_sh.at[pl.ds(base + nfull * CHUNK, rem)])

    @pl.when(sid == 0)
    def _():
        pltpu.sync_copy(rowsv.at[pl.ds(0, OUT_ROWS_REM)],
                        out_sh.at[pl.ds(OUT_REM_BASE, OUT_ROWS_REM)])
        pltpu.sync_copy(denv, den_sh)

    plsc.subcore_barrier()

    pltpu.sync_copy(as_hbm, asv)
    pltpu.sync_copy(ad_hbm, adv)
    pltpu.sync_copy(src_hbm.at[pl.ds(sid * EDGES_TILE, EDGES_TILE)], srcv)
    pltpu.sync_copy(dst_p1_hbm.at[sid], dstv2)

    def _edge_p(s16, d16):
        a1 = plsc.load_gather(asv, [s16])
        a2 = plsc.load_gather(adv, [d16])
        e = a1 + a2
        w = jnp.where(e >= 0.0, e, e * _NEG_SLOPE)
        return jnp.exp(w)

    def _p1(r, carry):
        for g in range(CHUNK // L):
            s16 = srcv[pl.ds(r * CHUNK + g * L, L)]
            d16 = dstv2[r, pl.ds(g * L, L)]
            pbuf[pl.ds(g * L, L)] = _edge_p(s16, d16)
        pltpu.sync_copy(pbuf, den_sh.at[dstv2.at[r]], add=True)
        return carry
    lax.fori_loop(0, ROWS, _p1, 0)

    plsc.subcore_barrier()

    pltpu.sync_copy(den_sh, denv)

    def _p2(r, carry):
        @pl.when(cid == 0)
        def _():
            pltpu.sync_copy(hlo_hbm.at[srcv.at[pl.ds(r * CHUNK, CHUNK)]],
                            rowsv)

        @pl.when(cid == 1)
        def _():
            pltpu.sync_copy(hhi_hbm.at[srcv.at[pl.ds(r * CHUNK, CHUNK)]],
                            rowsv)

        for g in range(CHUNK // L):
            s16 = srcv[pl.ds(r * CHUNK + g * L, L)]
            d16 = dstv2[r, pl.ds(g * L, L)]
            p = _edge_p(s16, d16)
            dv = plsc.load_gather(denv, [d16])
            pbuf[pl.ds(g * L, L)] = p / (dv + 1e-16)

        def _scale(j, c2):
            c16 = plsc.load_gather(pbuf, [jnp.full((L,), j, jnp.int32)])
            for q in range(QS):
                rowsv[j, pl.ds(q * L, L)] = rowsv[j, pl.ds(q * L, L)] * c16
            return c2
        lax.fori_loop(0, CHUNK, _scale, 0)

        pltpu.sync_copy(rowsv, out_sh.at[dstv2.at[r]], add=True)
        return carry
    lax.fori_loop(0, ROWS, _p2, 0)

    plsc.subcore_barrier()

    pltpu.sync_copy(out_sh.at[pl.ds(base, OUT_ROWS_TILE)],
                    out_hbm.at[cid].at[pl.ds(base, OUT_ROWS_TILE)])

    @pl.when(sid == 0)
    def _():
        pltpu.sync_copy(out_sh.at[pl.ds(OUT_REM_BASE, OUT_ROWS_REM)],
                        out_hbm.at[cid].at[pl.ds(OUT_REM_BASE, OUT_ROWS_REM)])


@functools.partial(
    pl.kernel,
    out_type=jax.ShapeDtypeStruct((C,), jnp.float32),
    mesh=_mesh,
    compiler_params=_sc_params,
    scratch_types=[
        pltpu.VMEM((C,), jnp.int32),
        pltpu.VMEM((C, DH), jnp.float32),
        pltpu.VMEM((C, DH), jnp.float32),
        pltpu.VMEM((D,), jnp.float32),
        pltpu.VMEM((C,), jnp.float32),
    ],
)
def _score_kernel(o_hbm, cand_hbm, w_hbm, out_hbm, candv, r0, r1, wv, sv):
    cid = lax.axis_index("c")
    sid = lax.axis_index("s")

    @pl.when((cid == 0) & (sid == 0))
    def _():
        pltpu.sync_copy(cand_hbm, candv)
        pltpu.sync_copy(w_hbm, wv)
        pltpu.sync_copy(o_hbm.at[0].at[candv], r0)
        pltpu.sync_copy(o_hbm.at[1].at[candv], r1)

        def _b16(v):
            u = plsc.bitcast(v, jnp.int32)
            r = u + 0x7FFF + ((u >> 16) & 1)
            return plsc.bitcast(r & jnp.int32(-65536), jnp.float32)

        for g in range(C // L):
            rid = lax.iota(jnp.int32, L) + g * L

            def _f(f, acc):
                f16 = jnp.full((L,), f, jnp.int32)
                c0 = _b16(jnp.maximum(plsc.load_gather(r0, [rid, f16]), 0.0))
                c1 = _b16(jnp.maximum(plsc.load_gather(r1, [rid, f16]), 0.0))
                w0 = _b16(plsc.load_gather(wv, [f16]))
                w1 = _b16(plsc.load_gather(wv, [f16 + DH]))
                return acc + c0 * w0 + c1 * w1
            acc = lax.fori_loop(0, DH, _f, jnp.zeros((L,), jnp.float32))
            sv[pl.ds(g * L, L)] = acc
        pltpu.sync_copy(sv, out_hbm)


def kernel(x, edge_index, cand_idxs, W1, a_src1, a_dst1,
           W2, a_src2, a_dst2, w_score, b_score):
    src = edge_index[0]
    dst = edge_index[1]
    dst_p1 = dst.reshape(NS, ROWS, CHUNK)

    hlo1, hhi1, as1, ad1 = _dense1(x, W1, a_src1, a_dst1)
    out1 = _edge_kernel(as1, ad1, hlo1, hhi1, src, dst_p1)
    hlo2, hhi2, as2, ad2 = _dense2(out1, W2, a_src2, a_dst2)
    out2 = _edge_kernel(as2, ad2, hlo2, hhi2, src, dst_p1)
    scores = _score_kernel(out2, cand_idxs, w_score)
    return scores + b_score


# speedup vs baseline: 27.1787x; 1.4550x over previous
import functools

import jax
import jax.numpy as jnp
from jax import lax
from jax.experimental import pallas as pl
from jax.experimental.pallas import tpu as pltpu
from jax.experimental.pallas import tpu_sc as plsc

N = 10000
E = 320000
D = 128
DH = D // 2
C = 32

L = 16
NC = 2
NS = 16
CHUNK = 80

EDGES_TILE = E // NS
ROWS = EDGES_TILE // CHUNK
QS = DH // L
OUT_ROWS_TILE = 624
OUT_ROWS_REM = N - NS * OUT_ROWS_TILE
OUT_REM_BASE = NS * OUT_ROWS_TILE

_NEG_SLOPE = 0.2


def _dense_common(x, W, a_s, a_d, hs_ref, als_ref, ald_ref):
    h = jnp.dot(x.astype(jnp.bfloat16), W.astype(jnp.bfloat16),
                preferred_element_type=jnp.float32)
    hs_ref[0] = h[:, :DH]
    hs_ref[1] = h[:, DH:]
    h16 = h.astype(jnp.bfloat16)
    als_ref[...] = jnp.dot(h16, a_s.astype(jnp.bfloat16),
                           preferred_element_type=jnp.float32)
    ald_ref[...] = jnp.dot(h16, a_d.astype(jnp.bfloat16),
                           preferred_element_type=jnp.float32)


def _dense1_body(x_ref, w_ref, as_ref, ad_ref, hs_ref, als_ref, ald_ref):
    _dense_common(x_ref[...], w_ref[...], as_ref[...], ad_ref[...],
                  hs_ref, als_ref, ald_ref)


def _dense2_body(o_ref, w_ref, as_ref, ad_ref, hs_ref, als_ref, ald_ref):
    t = jnp.maximum(jnp.concatenate([o_ref[0], o_ref[1]], axis=-1), 0.0)
    _dense_common(t, w_ref[...], as_ref[...], ad_ref[...],
                  hs_ref, als_ref, ald_ref)


_dense_out = [
    jax.ShapeDtypeStruct((NC, N, DH), jnp.float32),
    jax.ShapeDtypeStruct((N,), jnp.float32),
    jax.ShapeDtypeStruct((N,), jnp.float32),
]

_dense1 = pl.pallas_call(_dense1_body, out_shape=_dense_out)
_dense2 = pl.pallas_call(_dense2_body, out_shape=_dense_out)


_mesh = plsc.VectorSubcoreMesh(core_axis_name="c", subcore_axis_name="s")
_sc_params = pltpu.CompilerParams(needs_layout_passes=False,
                                  use_tc_tiling_on_sc=False)


@functools.partial(
    pl.kernel,
    out_type=jax.ShapeDtypeStruct((NC, N, DH), jnp.float32),
    mesh=_mesh,
    compiler_params=_sc_params,
    scratch_types=[
        pltpu.VMEM((N,), jnp.float32),
        pltpu.VMEM((N,), jnp.float32),
        pltpu.VMEM((N,), jnp.float32),
        pltpu.VMEM((EDGES_TILE,), jnp.int32),
        pltpu.VMEM((ROWS, CHUNK), jnp.int32),
        pltpu.VMEM((CHUNK, DH), jnp.float32),
        pltpu.VMEM((CHUNK, DH), jnp.float32),
        pltpu.VMEM((5, CHUNK), jnp.float32),
        pltpu.VMEM((CHUNK,), jnp.float32),
        pltpu.VMEM_SHARED((N,), jnp.float32),
        pltpu.VMEM_SHARED((N, DH), jnp.float32),
        pltpu.SemaphoreType.DMA,
        pltpu.SemaphoreType.DMA,
        pltpu.SemaphoreType.DMA,
        pltpu.SemaphoreType.DMA,
        pltpu.SemaphoreType.DMA,
    ],
)
def _edge_kernel(as_hbm, ad_hbm, h_hbm, src_hbm, dst_p1_hbm,
                 out_hbm, asv, adv, denv, srcv, dstv2, rowsa, rowsb,
                 pbufs, pbuf, den_sh, out_sh, s1, ga, gb, sa, sb):
    cid = lax.axis_index("c")
    sid = lax.axis_index("s")
    zero16 = jnp.zeros((L,), jnp.float32)

    def _zrow(r, carry):
        for q in range(QS):
            rowsa[r, pl.ds(q * L, L)] = zero16
        return carry
    lax.fori_loop(0, CHUNK, _zrow, 0)

    def _zden(i, carry):
        denv[pl.ds(i * L, L)] = zero16
        return carry
    lax.fori_loop(0, N // L, _zden, 0)

    base = sid * OUT_ROWS_TILE
    nfull = OUT_ROWS_TILE // CHUNK
    rem = OUT_ROWS_TILE - nfull * CHUNK
    for k in range(nfull):
        pltpu.sync_copy(rowsa, out_sh.at[pl.ds(base + k * CHUNK, CHUNK)])
    pltpu.sync_copy(rowsa.at[pl.ds(0, rem)],
                    out_sh.at[pl.ds(base + nfull * CHUNK, rem)])

    @pl.when(sid == 0)
    def _():
        pltpu.sync_copy(rowsa.at[pl.ds(0, OUT_ROWS_REM)],
                        out_sh.at[pl.ds(OUT_REM_BASE, OUT_ROWS_REM)])
        pltpu.sync_copy(denv, den_sh)

    plsc.subcore_barrier()

    pltpu.sync_copy(as_hbm, asv)
    pltpu.sync_copy(ad_hbm, adv)
    pltpu.sync_copy(src_hbm.at[pl.ds(sid * EDGES_TILE, EDGES_TILE)], srcv)
    pltpu.sync_copy(dst_p1_hbm.at[sid], dstv2)

    def _edge_p(s16, d16):
        a1 = plsc.load_gather(asv, [s16])
        a2 = plsc.load_gather(adv, [d16])
        e = a1 + a2
        w = jnp.where(e >= 0.0, e, e * _NEG_SLOPE)
        return jnp.exp(w)

    def _p1(t, carry):
        handles = []
        for j in range(5):
            r = t * 5 + j
            for g in range(CHUNK // L):
                s16 = srcv[pl.ds(r * CHUNK + g * L, L)]
                d16 = dstv2[r, pl.ds(g * L, L)]
                pbufs[j, pl.ds(g * L, L)] = _edge_p(s16, d16)
            handles.append(pltpu.async_copy(
                pbufs.at[j], den_sh.at[dstv2.at[r]], s1, add=True))
        for hnd in handles:
            hnd.wait()
        return carry
    lax.fori_loop(0, ROWS // 5, _p1, 0)

    plsc.subcore_barrier()

    pltpu.sync_copy(den_sh, denv)

    def _issue_gather(r, buf, sem):
        pltpu.async_copy(
            h_hbm.at[cid].at[srcv.at[pl.ds(r * CHUNK, CHUNK)]], buf, sem)

    def _drain(buf, sem):
        pltpu.make_async_copy(h_hbm.at[0].at[pl.ds(0, CHUNK)], buf, sem).wait()

    def _coef_scale(r, rows):
        for g in range(CHUNK // L):
            s16 = srcv[pl.ds(r * CHUNK + g * L, L)]
            d16 = dstv2[r, pl.ds(g * L, L)]
            p = _edge_p(s16, d16)
            dv = plsc.load_gather(denv, [d16])
            pbuf[pl.ds(g * L, L)] = p / (dv + 1e-16)

        def _scale(j, c2):
            c16 = plsc.load_gather(pbuf, [jnp.full((L,), j, jnp.int32)])
            for q in range(QS):
                rows[j, pl.ds(q * L, L)] = rows[j, pl.ds(q * L, L)] * c16
            return c2
        lax.fori_loop(0, CHUNK, _scale, 0)

    _issue_gather(0, rowsa, ga)
    _issue_gather(1, rowsb, gb)

    def _p2(k, carry):
        r0 = 2 * k
        r1 = 2 * k + 1
        _drain(rowsa, ga)
        _coef_scale(r0, rowsa)
        pltpu.async_copy(rowsa, out_sh.at[dstv2.at[r0]], sa, add=True)
        _drain(rowsb, gb)
        _coef_scale(r1, rowsb)
        pltpu.async_copy(rowsb, out_sh.at[dstv2.at[r1]], sb, add=True)
        _drain(rowsa, sa)
        _issue_gather(jnp.minimum(r0 + 2, ROWS - 1), rowsa, ga)
        _drain(rowsb, sb)
        _issue_gather(jnp.minimum(r1 + 2, ROWS - 1), rowsb, gb)
        return carry
    lax.fori_loop(0, ROWS // 2, _p2, 0)

    _drain(rowsa, ga)
    _drain(rowsb, gb)

    plsc.subcore_barrier()

    pltpu.sync_copy(out_sh.at[pl.ds(base, OUT_ROWS_TILE)],
                    out_hbm.at[cid].at[pl.ds(base, OUT_ROWS_TILE)])

    @pl.when(sid == 0)
    def _():
        pltpu.sync_copy(out_sh.at[pl.ds(OUT_REM_BASE, OUT_ROWS_REM)],
                        out_hbm.at[cid].at[pl.ds(OUT_REM_BASE, OUT_ROWS_REM)])


@functools.partial(
    pl.kernel,
    out_type=jax.ShapeDtypeStruct((C,), jnp.float32),
    mesh=_mesh,
    compiler_params=_sc_params,
    scratch_types=[
        pltpu.VMEM((C,), jnp.int32),
        pltpu.VMEM((C, DH), jnp.float32),
        pltpu.VMEM((C, DH), jnp.float32),
        pltpu.VMEM((D,), jnp.float32),
        pltpu.VMEM((C,), jnp.float32),
    ],
)
def _score_kernel(o_hbm, cand_hbm, w_hbm, out_hbm, candv, r0, r1, wv, sv):
    cid = lax.axis_index("c")
    sid = lax.axis_index("s")

    @pl.when((cid == 0) & (sid == 0))
    def _():
        pltpu.sync_copy(cand_hbm, candv)
        pltpu.sync_copy(w_hbm, wv)
        pltpu.sync_copy(o_hbm.at[0].at[candv], r0)
        pltpu.sync_copy(o_hbm.at[1].at[candv], r1)

        def _b16(v):
            u = plsc.bitcast(v, jnp.int32)
            r = u + 0x7FFF + ((u >> 16) & 1)
            return plsc.bitcast(r & jnp.int32(-65536), jnp.float32)

        for g in range(C // L):
            rid = lax.iota(jnp.int32, L) + g * L

            def _f(f, acc):
                f16 = jnp.full((L,), f, jnp.int32)
                c0 = _b16(jnp.maximum(plsc.load_gather(r0, [rid, f16]), 0.0))
                c1 = _b16(jnp.maximum(plsc.load_gather(r1, [rid, f16]), 0.0))
                w0 = _b16(plsc.load_gather(wv, [f16]))
                w1 = _b16(plsc.load_gather(wv, [f16 + DH]))
                return acc + c0 * w0 + c1 * w1
            acc = lax.fori_loop(0, DH, _f, jnp.zeros((L,), jnp.float32))
            sv[pl.ds(g * L, L)] = acc
        pltpu.sync_copy(sv, out_hbm)


def kernel(x, edge_index, cand_idxs, W1, a_src1, a_dst1,
           W2, a_src2, a_dst2, w_score, b_score):
    src = edge_index[0]
    dst = edge_index[1]
    dst_p1 = dst.reshape(NS, ROWS, CHUNK)

    hs1, as1, ad1 = _dense1(x, W1, a_src1, a_dst1)
    out1 = _edge_kernel(as1, ad1, hs1, src, dst_p1)
    hs2, as2, ad2 = _dense2(out1, W2, a_src2, a_dst2)
    out2 = _edge_kernel(as2, ad2, hs2, src, dst_p1)
    scores = _score_kernel(out2, cand_idxs, w_score)
    return scores + b_score


# speedup vs baseline: 28.3137x; 1.0418x over previous
import functools

import jax
import jax.numpy as jnp
from jax import lax
from jax.experimental import pallas as pl
from jax.experimental.pallas import tpu as pltpu
from jax.experimental.pallas import tpu_sc as plsc

N = 10000
E = 320000
D = 128
DH = D // 2
C = 32

L = 16
NC = 2
NS = 16
CHUNK = 80

EDGES_TILE = E // NS
ROWS = EDGES_TILE // CHUNK
QS = DH // L
OUT_ROWS_TILE = 624
OUT_ROWS_REM = N - NS * OUT_ROWS_TILE
OUT_REM_BASE = NS * OUT_ROWS_TILE

_NEG_SLOPE = 0.2


def _dense_common(x, W, a_s, a_d, hs_ref, als_ref, ald_ref):
    h = jnp.dot(x.astype(jnp.bfloat16), W.astype(jnp.bfloat16),
                preferred_element_type=jnp.float32)
    hs_ref[0] = h[:, :DH]
    hs_ref[1] = h[:, DH:]
    h16 = h.astype(jnp.bfloat16)
    als_ref[...] = jnp.dot(h16, a_s.astype(jnp.bfloat16),
                           preferred_element_type=jnp.float32)
    ald_ref[...] = jnp.dot(h16, a_d.astype(jnp.bfloat16),
                           preferred_element_type=jnp.float32)


def _dense1_body(x_ref, w_ref, as_ref, ad_ref, hs_ref, als_ref, ald_ref):
    _dense_common(x_ref[...], w_ref[...], as_ref[...], ad_ref[...],
                  hs_ref, als_ref, ald_ref)


def _dense2_body(o_ref, w_ref, as_ref, ad_ref, hs_ref, als_ref, ald_ref):
    t = jnp.maximum(jnp.concatenate([o_ref[0], o_ref[1]], axis=-1), 0.0)
    _dense_common(t, w_ref[...], as_ref[...], ad_ref[...],
                  hs_ref, als_ref, ald_ref)


_dense_out = [
    jax.ShapeDtypeStruct((NC, N, DH), jnp.float32),
    jax.ShapeDtypeStruct((N,), jnp.float32),
    jax.ShapeDtypeStruct((N,), jnp.float32),
]

_dense1 = pl.pallas_call(_dense1_body, out_shape=_dense_out)
_dense2 = pl.pallas_call(_dense2_body, out_shape=_dense_out)


_mesh = plsc.VectorSubcoreMesh(core_axis_name="c", subcore_axis_name="s")
_sc_params = pltpu.CompilerParams(needs_layout_passes=False,
                                  use_tc_tiling_on_sc=False)


@functools.partial(
    pl.kernel,
    out_type=jax.ShapeDtypeStruct((NC, N, DH), jnp.float32),
    mesh=_mesh,
    compiler_params=_sc_params,
    scratch_types=[
        pltpu.VMEM((N,), jnp.float32),
        pltpu.VMEM((N,), jnp.float32),
        pltpu.VMEM((N,), jnp.float32),
        pltpu.VMEM((EDGES_TILE,), jnp.int32),
        pltpu.VMEM((ROWS, CHUNK), jnp.int32),
        pltpu.VMEM((CHUNK, DH), jnp.float32),
        pltpu.VMEM((CHUNK, DH), jnp.float32),
        pltpu.VMEM((5, CHUNK), jnp.float32),
        pltpu.VMEM((CHUNK,), jnp.float32),
        pltpu.VMEM_SHARED((N,), jnp.float32),
        pltpu.VMEM_SHARED((N, DH), jnp.float32),
        pltpu.SemaphoreType.DMA,
        pltpu.SemaphoreType.DMA,
        pltpu.SemaphoreType.DMA,
        pltpu.SemaphoreType.DMA,
        pltpu.SemaphoreType.DMA,
    ],
)
def _edge_kernel(as_hbm, ad_hbm, h_hbm, src_hbm, dst_p1_hbm,
                 out_hbm, asv, adv, denv, srcv, dstv2, rowsa, rowsb,
                 pbufs, pbuf, den_sh, out_sh, s1, ga, gb, sa, sb):
    cid = lax.axis_index("c")
    sid = lax.axis_index("s")
    zero16 = jnp.zeros((L,), jnp.float32)

    def _zrow(r, carry):
        for q in range(QS):
            rowsa[r, pl.ds(q * L, L)] = zero16
        return carry
    lax.fori_loop(0, CHUNK, _zrow, 0)

    def _zden(i, carry):
        denv[pl.ds(i * L, L)] = zero16
        return carry
    lax.fori_loop(0, N // L, _zden, 0)

    base = sid * OUT_ROWS_TILE
    nfull = OUT_ROWS_TILE // CHUNK
    rem = OUT_ROWS_TILE - nfull * CHUNK
    for k in range(nfull):
        pltpu.sync_copy(rowsa, out_sh.at[pl.ds(base + k * CHUNK, CHUNK)])
    pltpu.sync_copy(rowsa.at[pl.ds(0, rem)],
                    out_sh.at[pl.ds(base + nfull * CHUNK, rem)])

    @pl.when(sid == 0)
    def _():
        pltpu.sync_copy(rowsa.at[pl.ds(0, OUT_ROWS_REM)],
                        out_sh.at[pl.ds(OUT_REM_BASE, OUT_ROWS_REM)])
        pltpu.sync_copy(denv, den_sh)

    plsc.subcore_barrier()

    pltpu.sync_copy(as_hbm, asv)
    pltpu.sync_copy(ad_hbm, adv)
    pltpu.sync_copy(src_hbm.at[pl.ds(sid * EDGES_TILE, EDGES_TILE)], srcv)
    pltpu.sync_copy(dst_p1_hbm.at[sid], dstv2)

    def _edge_p(s16, d16):
        a1 = plsc.load_gather(asv, [s16])
        a2 = plsc.load_gather(adv, [d16])
        e = a1 + a2
        w = jnp.where(e >= 0.0, e, e * _NEG_SLOPE)
        return jnp.exp(w)

    def _p1(t, carry):
        handles = []
        for j in range(5):
            r = t * 5 + j
            for g in range(CHUNK // L):
                s16 = srcv[pl.ds(r * CHUNK + g * L, L)]
                d16 = dstv2[r, pl.ds(g * L, L)]
                pbufs[j, pl.ds(g * L, L)] = _edge_p(s16, d16)
            handles.append(pltpu.async_copy(
                pbufs.at[j], den_sh.at[dstv2.at[r]], s1, add=True))
        for hnd in handles:
            hnd.wait()
        return carry
    lax.fori_loop(0, ROWS // 5, _p1, 0)

    plsc.subcore_barrier()

    pltpu.sync_copy(den_sh, denv)

    def _issue_gather(r, buf, sem):
        pltpu.async_copy(
            h_hbm.at[cid].at[srcv.at[pl.ds(r * CHUNK, CHUNK)]], buf, sem)

    def _drain(buf, sem):
        pltpu.make_async_copy(h_hbm.at[0].at[pl.ds(0, CHUNK)], buf, sem).wait()

    def _coef_scale(r, rows):
        for g in range(CHUNK // L):
            s16 = srcv[pl.ds(r * CHUNK + g * L, L)]
            d16 = dstv2[r, pl.ds(g * L, L)]
            p = _edge_p(s16, d16)
            dv = plsc.load_gather(denv, [d16])
            pbuf[pl.ds(g * L, L)] = p / (dv + 1e-16)

        def _scale(jj, c2):
            j0 = jj * 8
            for u in range(8):
                j = j0 + u
                c16 = plsc.load_gather(pbuf, [jnp.full((L,), j, jnp.int32)])
                for q in range(QS):
                    rows[j, pl.ds(q * L, L)] = rows[j, pl.ds(q * L, L)] * c16
            return c2
        lax.fori_loop(0, CHUNK // 8, _scale, 0)

    _issue_gather(0, rowsa, ga)
    _issue_gather(1, rowsb, gb)

    def _p2(k, carry):
        r0 = 2 * k
        r1 = 2 * k + 1
        _drain(rowsa, ga)
        _coef_scale(r0, rowsa)
        pltpu.async_copy(rowsa, out_sh.at[dstv2.at[r0]], sa, add=True)
        _drain(rowsb, gb)
        _coef_scale(r1, rowsb)
        pltpu.async_copy(rowsb, out_sh.at[dstv2.at[r1]], sb, add=True)
        _drain(rowsa, sa)
        _issue_gather(jnp.minimum(r0 + 2, ROWS - 1), rowsa, ga)
        _drain(rowsb, sb)
        _issue_gather(jnp.minimum(r1 + 2, ROWS - 1), rowsb, gb)
        return carry
    lax.fori_loop(0, ROWS // 2, _p2, 0)

    _drain(rowsa, ga)
    _drain(rowsb, gb)

    plsc.subcore_barrier()

    pltpu.sync_copy(out_sh.at[pl.ds(base, OUT_ROWS_TILE)],
                    out_hbm.at[cid].at[pl.ds(base, OUT_ROWS_TILE)])

    @pl.when(sid == 0)
    def _():
        pltpu.sync_copy(out_sh.at[pl.ds(OUT_REM_BASE, OUT_ROWS_REM)],
                        out_hbm.at[cid].at[pl.ds(OUT_REM_BASE, OUT_ROWS_REM)])


@functools.partial(
    pl.kernel,
    out_type=jax.ShapeDtypeStruct((C,), jnp.float32),
    mesh=_mesh,
    compiler_params=_sc_params,
    scratch_types=[
        pltpu.VMEM((C,), jnp.int32),
        pltpu.VMEM((C, DH), jnp.float32),
        pltpu.VMEM((C, DH), jnp.float32),
        pltpu.VMEM((D,), jnp.float32),
        pltpu.VMEM((C,), jnp.float32),
    ],
)
def _score_kernel(o_hbm, cand_hbm, w_hbm, out_hbm, candv, r0, r1, wv, sv):
    cid = lax.axis_index("c")
    sid = lax.axis_index("s")

    @pl.when((cid == 0) & (sid == 0))
    def _():
        pltpu.sync_copy(cand_hbm, candv)
        pltpu.sync_copy(w_hbm, wv)
        pltpu.sync_copy(o_hbm.at[0].at[candv], r0)
        pltpu.sync_copy(o_hbm.at[1].at[candv], r1)

        def _b16(v):
            u = plsc.bitcast(v, jnp.int32)
            r = u + 0x7FFF + ((u >> 16) & 1)
            return plsc.bitcast(r & jnp.int32(-65536), jnp.float32)

        for g in range(C // L):
            rid = lax.iota(jnp.int32, L) + g * L

            def _f(f, acc):
                f16 = jnp.full((L,), f, jnp.int32)
                c0 = _b16(jnp.maximum(plsc.load_gather(r0, [rid, f16]), 0.0))
                c1 = _b16(jnp.maximum(plsc.load_gather(r1, [rid, f16]), 0.0))
                w0 = _b16(plsc.load_gather(wv, [f16]))
                w1 = _b16(plsc.load_gather(wv, [f16 + DH]))
                return acc + c0 * w0 + c1 * w1
            acc = lax.fori_loop(0, DH, _f, jnp.zeros((L,), jnp.float32))
            sv[pl.ds(g * L, L)] = acc
        pltpu.sync_copy(sv, out_hbm)


def kernel(x, edge_index, cand_idxs, W1, a_src1, a_dst1,
           W2, a_src2, a_dst2, w_score, b_score):
    src = edge_index[0]
    dst = edge_index[1]
    dst_p1 = dst.reshape(NS, ROWS, CHUNK)

    hs1, as1, ad1 = _dense1(x, W1, a_src1, a_dst1)
    out1 = _edge_kernel(as1, ad1, hs1, src, dst_p1)
    hs2, as2, ad2 = _dense2(out1, W2, a_src2, a_dst2)
    out2 = _edge_kernel(as2, ad2, hs2, src, dst_p1)
    scores = _score_kernel(out2, cand_idxs, w_score)
    return scores + b_score
